# baseline (speedup 1.0000x reference)
import functools

import jax
import jax.numpy as jnp
from jax.experimental import pallas as pl
from jax.experimental.pallas import tpu as pltpu

N = 10000
D = 512
H = 8
NSTR = 2
BLK = 1000
GRID = N // (NSTR * BLK)


def _body(*refs):
    x_refs = refs[:NSTR]
    w_ref, b_ref, o_ref, c_ref, m_ref, s_ref, acc_ref = refs[NSTR:]
    i = pl.program_id(0)

    @pl.when(i == 0)
    def _init():
        x0 = x_refs[0][0:1, :]
        w1 = w_ref[:, :D]
        c_ref[...] = jax.lax.dot_general(
            x0, w1, (((1,), (1,)), ((), ())),
            preferred_element_type=jnp.float32) + b_ref[...]
        m_ref[...] = jnp.full_like(m_ref, -jnp.inf)
        s_ref[...] = jnp.zeros_like(s_ref)
        acc_ref[...] = jnp.zeros_like(acc_ref)

    w2 = w_ref[:, D:]
    xs = [r[...] for r in x_refs]
    w2b = w2.astype(jnp.bfloat16)
    ls = []
    for x in xs:
        l = jax.lax.dot_general(
            x.astype(jnp.bfloat16), w2b, (((1,), (1,)), ((), ())),
            preferred_element_type=jnp.float32) + c_ref[...]
        ls.append(jnp.where(l >= 0, l, 0.01 * l))

    m_old = m_ref[...]
    m_blk = ls[0].max(axis=0, keepdims=True)
    for l in ls[1:]:
        m_blk = jnp.maximum(m_blk, l.max(axis=0, keepdims=True))
    m_new = jnp.maximum(m_old, m_blk)
    r = jnp.exp(m_old - m_new)
    m_ref[...] = m_new

    ps = [jnp.exp(l - m_new) for l in ls]
    s_new = s_ref[...] * r
    for p in ps:
        s_new = s_new + jnp.sum(p, axis=0, keepdims=True)
    s_ref[...] = s_new

    a_new = acc_ref[...] * r.T
    for p, x in zip(ps, xs):
        a_new = a_new + jax.lax.dot_general(
            p.astype(jnp.bfloat16), x.astype(jnp.bfloat16),
            (((0,), (0,)), ((), ())),
            preferred_element_type=jnp.float32)
    acc_ref[...] = a_new

    @pl.when(i == pl.num_programs(0) - 1)
    def _fin():
        head_avg = jnp.sum(acc_ref[...] / s_ref[...].T, axis=0,
                           keepdims=True) / H
        o_ref[...] = jnp.maximum(head_avg, 0.0)


def _stream_spec(k):
    return pl.BlockSpec((BLK, D), lambda i, k=k: (k * GRID + i, 0))


@jax.jit
def _run(attention_mx, W, b):
    out = pl.pallas_call(
        _body,
        grid=(GRID,),
        in_specs=[_stream_spec(k) for k in range(NSTR)] + [
            pl.BlockSpec((H, 2 * D), lambda i: (0, 0)),
            pl.BlockSpec((1, H), lambda i: (0, 0)),
        ],
        out_specs=pl.BlockSpec((1, D), lambda i: (0, 0)),
        out_shape=jax.ShapeDtypeStruct((1, D), jnp.float32),
        scratch_shapes=[
            pltpu.VMEM((1, H), jnp.float32),
            pltpu.VMEM((1, H), jnp.float32),
            pltpu.VMEM((1, H), jnp.float32),
            pltpu.VMEM((H, D), jnp.float32),
        ],
    )(*([attention_mx] * NSTR), W, b.reshape(1, H))
    return out.reshape(D)


def kernel(attention_mx, W, b):
    return _run(attention_mx, W, b)


# speedup vs baseline: 1.0132x; 1.0132x over previous
import functools

import jax
import jax.numpy as jnp
from jax.experimental import pallas as pl
from jax.experimental.pallas import tpu as pltpu

N = 10000
D = 512
H = 8
NSTR = 2
BLK = 1000
GRID = N // (NSTR * BLK)


def _body(*refs):
    x_refs = refs[:NSTR]
    w_ref, b_ref, o_ref, c_ref, m_ref, s_ref, acc_ref = refs[NSTR:]
    i = pl.program_id(0)

    @pl.when(i == 0)
    def _init():
        x0 = x_refs[0][0:1, :]
        w1 = w_ref[:, :D]
        c_ref[...] = jax.lax.dot_general(
            x0, w1, (((1,), (1,)), ((), ())),
            preferred_element_type=jnp.float32) + b_ref[...]
        m_ref[...] = jnp.full_like(m_ref, -jnp.inf)
        s_ref[...] = jnp.zeros_like(s_ref)
        acc_ref[...] = jnp.zeros_like(acc_ref)

    w2 = w_ref[:, D:]
    xs = [r[...] for r in x_refs]
    ls = []
    for x in xs:
        l = jax.lax.dot_general(
            x, w2, (((1,), (1,)), ((), ())),
            preferred_element_type=jnp.float32) + c_ref[...]
        ls.append(jnp.where(l >= 0, l, 0.01 * l))

    m_old = m_ref[...]
    m_blk = ls[0].max(axis=0, keepdims=True)
    for l in ls[1:]:
        m_blk = jnp.maximum(m_blk, l.max(axis=0, keepdims=True))
    m_new = jnp.maximum(m_old, m_blk)
    r = jnp.exp(m_old - m_new)
    m_ref[...] = m_new

    ps = [jnp.exp(l - m_new) for l in ls]
    s_new = s_ref[...] * r
    for p in ps:
        s_new = s_new + jnp.sum(p, axis=0, keepdims=True)
    s_ref[...] = s_new

    a_new = acc_ref[...] * r.T
    for p, x in zip(ps, xs):
        a_new = a_new + jax.lax.dot_general(
            p, x, (((0,), (0,)), ((), ())),
            preferred_element_type=jnp.float32)
    acc_ref[...] = a_new

    @pl.when(i == pl.num_programs(0) - 1)
    def _fin():
        head_avg = jnp.sum(acc_ref[...] / s_ref[...].T, axis=0,
                           keepdims=True) / H
        o_ref[...] = jnp.maximum(head_avg, 0.0)


def _stream_spec(k):
    return pl.BlockSpec((BLK, D), lambda i, k=k: (k * GRID + i, 0))


@jax.jit
def _run(attention_mx, W, b):
    out = pl.pallas_call(
        _body,
        grid=(GRID,),
        in_specs=[_stream_spec(k) for k in range(NSTR)] + [
            pl.BlockSpec((H, 2 * D), lambda i: (0, 0)),
            pl.BlockSpec((1, H), lambda i: (0, 0)),
        ],
        out_specs=pl.BlockSpec((1, D), lambda i: (0, 0)),
        out_shape=jax.ShapeDtypeStruct((1, D), jnp.float32),
        scratch_shapes=[
            pltpu.VMEM((1, H), jnp.float32),
            pltpu.VMEM((1, H), jnp.float32),
            pltpu.VMEM((1, H), jnp.float32),
            pltpu.VMEM((H, D), jnp.float32),
        ],
    )(*([attention_mx] * NSTR), W, b.reshape(1, H))
    return out.reshape(D)


def kernel(attention_mx, W, b):
    return _run(attention_mx, W, b)
